# baseline (speedup 1.0000x reference)
import functools

import jax
import jax.numpy as jnp
from jax import lax
from jax.experimental import pallas as pl
from jax.experimental.pallas import tpu as pltpu
from jax.experimental.pallas import tpu_sc as plsc

M = 100000
D = 128
B = 16384
PARA = 0.5

NC, NS = 2, 16
NW = NC * NS
BPW = B // NW


@functools.cache
def _sc_kernels():
    mesh = plsc.VectorSubcoreMesh(
        core_axis_name="c", subcore_axis_name="s", num_cores=NC, num_subcores=NS
    )

    @functools.partial(
        pl.kernel,
        mesh=mesh,
        out_type=jax.ShapeDtypeStruct((B, D), jnp.float32),
        scratch_types=[
            [pltpu.VMEM((BPW // 4,), jnp.int32) for _ in range(4)],
            [pltpu.VMEM((BPW // 4, D), jnp.float32) for _ in range(4)],
            pltpu.VMEM((BPW,), jnp.int32),
            pltpu.VMEM((BPW,), jnp.int32),
            [pltpu.SemaphoreType.DMA for _ in range(4)],
            pltpu.SemaphoreType.DMA,
            pltpu.SemaphoreType.DMA,
        ],
    )
    def sc_gather(mem_hbm, idx_hbm, ts_hbm, lu_ref, out_hbm,
                  idx4, rows4, idx_v, ts_v, sem4, sem_t, sem_w):
        wid = lax.axis_index("s") * NC + lax.axis_index("c")
        base = wid * BPW
        Q = BPW // 4
        for j in range(4):
            pltpu.sync_copy(idx_hbm.at[pl.ds(base + j * Q, Q)], idx4[j])
        gathers = [
            pltpu.async_copy(mem_hbm.at[idx4[j]], rows4[j], sem4[j])
            for j in range(4)
        ]
        pltpu.sync_copy(idx_hbm.at[pl.ds(base, BPW)], idx_v)
        pltpu.sync_copy(ts_hbm.at[pl.ds(base, BPW)], ts_v)
        cp_ts = pltpu.async_copy(ts_v, lu_ref.at[idx_v], sem_t)
        writes = []
        for j in range(4):
            gathers[j].wait()
            writes.append(
                pltpu.async_copy(rows4[j], out_hbm.at[pl.ds(base + j * Q, Q)], sem_w)
            )
        cp_ts.wait()
        for w in writes:
            w.wait()

    @functools.partial(
        pl.kernel,
        mesh=mesh,
        out_type=(),
        scratch_types=[
            pltpu.VMEM((BPW,), jnp.int32),
            pltpu.VMEM((BPW, D), jnp.float32),
            pltpu.SemaphoreType.DMA,
        ],
    )
    def sc_scatter(upd_hbm, idx_hbm, mem_ref, idx_v, rows_v, sem):
        wid = lax.axis_index("s") * NC + lax.axis_index("c")
        base = wid * BPW
        pltpu.sync_copy(idx_hbm.at[pl.ds(base, BPW)], idx_v)
        pltpu.sync_copy(upd_hbm.at[pl.ds(base, BPW)], rows_v)
        pltpu.async_copy(rows_v, mem_ref.at[idx_v], sem).wait()

    return sc_gather, sc_scatter


_CR = 1000


def _tc_copy_body(src_ref, dst_ref):
    dst_ref[...] = src_ref[...]


def _tc_copy(memory):
    return pl.pallas_call(
        _tc_copy_body,
        grid=(M // _CR,),
        in_specs=[pl.BlockSpec((_CR, D), lambda i: (i, 0))],
        out_specs=pl.BlockSpec((_CR, D), lambda i: (i, 0)),
        out_shape=jax.ShapeDtypeStruct((M, D), jnp.float32),
    )(memory)


_BM = 2048


def _tc_body(mem_ref, msg_ref, w1m_ref, w1c_ref, w2_ref, out_ref):
    msg = msg_ref[...]
    mem = mem_ref[...]
    z = jnp.dot(msg, w1m_ref[...], preferred_element_type=jnp.float32)
    z = z + jnp.dot(mem, w1c_ref[...], preferred_element_type=jnp.float32)
    w = jnp.maximum(jnp.tanh(z), 0.0) * PARA
    u = jnp.tanh(jnp.dot(msg, w2_ref[...], preferred_element_type=jnp.float32))
    out_ref[...] = mem * (1.0 - w) + w * u


def _tc_update(mem_b, msgs, w1m, w1c, w2):
    return pl.pallas_call(
        _tc_body,
        grid=(B // _BM,),
        in_specs=[
            pl.BlockSpec((_BM, D), lambda i: (i, 0)),
            pl.BlockSpec((_BM, D), lambda i: (i, 0)),
            pl.BlockSpec((D, D), lambda i: (0, 0)),
            pl.BlockSpec((D, D), lambda i: (0, 0)),
            pl.BlockSpec((D, D), lambda i: (0, 0)),
        ],
        out_specs=pl.BlockSpec((_BM, D), lambda i: (i, 0)),
        out_shape=jax.ShapeDtypeStruct((B, D), jnp.float32),
    )(mem_b, msgs, w1m, w1c, w2)


def kernel(memory, unique_messages, W_lins, W_lin2, unique_node_ids, timestamps, last_update):
    sc_gather, sc_scatter = _sc_kernels()
    w1m = W_lins[:, :D].T
    w1c = W_lins[:, D:].T
    w2 = W_lin2.T

    lu_ref = jax.new_ref(last_update)
    mem_b = sc_gather(memory, unique_node_ids, timestamps, lu_ref)
    table = _tc_copy(memory)
    updated = _tc_update(mem_b, unique_messages, w1m, w1c, w2)

    tbl_ref = jax.new_ref(table)
    sc_scatter(updated, unique_node_ids, tbl_ref)
    return tbl_ref[...], lu_ref[...]


# speedup vs baseline: 1.2932x; 1.2932x over previous
import functools

import jax
import jax.numpy as jnp
from jax import lax
from jax.experimental import pallas as pl
from jax.experimental.pallas import tpu as pltpu
from jax.experimental.pallas import tpu_sc as plsc

M = 100000
D = 128
B = 16384
PARA = 0.5

NC, NS = 2, 16
NW = NC * NS
BPW = B // NW


@functools.cache
def _sc_kernels():
    mesh = plsc.VectorSubcoreMesh(
        core_axis_name="c", subcore_axis_name="s", num_cores=NC, num_subcores=NS
    )

    @functools.partial(
        pl.kernel,
        mesh=mesh,
        out_type=jax.ShapeDtypeStruct((B, D), jnp.float32),
        scratch_types=[
            [pltpu.VMEM((BPW // 4,), jnp.int32) for _ in range(4)],
            [pltpu.VMEM((BPW // 4, D), jnp.float32) for _ in range(4)],
            pltpu.VMEM((BPW,), jnp.int32),
            pltpu.VMEM((BPW,), jnp.int32),
            [pltpu.SemaphoreType.DMA for _ in range(4)],
            pltpu.SemaphoreType.DMA,
            pltpu.SemaphoreType.DMA,
        ],
    )
    def sc_gather(mem_hbm, idx_hbm, ts_hbm, lu_ref, out_hbm,
                  idx4, rows4, idx_v, ts_v, sem4, sem_t, sem_w):
        wid = lax.axis_index("s") * NC + lax.axis_index("c")
        base = wid * BPW
        Q = BPW // 4
        for j in range(4):
            pltpu.sync_copy(idx_hbm.at[pl.ds(base + j * Q, Q)], idx4[j])
        gathers = [
            pltpu.async_copy(mem_hbm.at[idx4[j]], rows4[j], sem4[j])
            for j in range(4)
        ]
        pltpu.sync_copy(idx_hbm.at[pl.ds(base, BPW)], idx_v)
        pltpu.sync_copy(ts_hbm.at[pl.ds(base, BPW)], ts_v)
        cp_ts = pltpu.async_copy(ts_v, lu_ref.at[idx_v], sem_t)
        writes = []
        for j in range(4):
            gathers[j].wait()
            writes.append(
                pltpu.async_copy(rows4[j], out_hbm.at[pl.ds(base + j * Q, Q)], sem_w)
            )
        cp_ts.wait()
        for w in writes:
            w.wait()

    @functools.partial(
        pl.kernel,
        mesh=mesh,
        out_type=(),
        scratch_types=[
            pltpu.VMEM((BPW,), jnp.int32),
            pltpu.VMEM((BPW, D), jnp.float32),
            pltpu.SemaphoreType.DMA,
        ],
    )
    def sc_scatter(upd_hbm, idx_hbm, mem_ref, idx_v, rows_v, sem):
        wid = lax.axis_index("s") * NC + lax.axis_index("c")
        base = wid * BPW
        pltpu.sync_copy(idx_hbm.at[pl.ds(base, BPW)], idx_v)
        pltpu.sync_copy(upd_hbm.at[pl.ds(base, BPW)], rows_v)
        pltpu.async_copy(rows_v, mem_ref.at[idx_v], sem).wait()

    return sc_gather, sc_scatter


_CR = 5000


def _tc_copy_body(src_ref, dst_ref):
    dst_ref[...] = src_ref[...]


def _tc_copy(memory):
    return pl.pallas_call(
        _tc_copy_body,
        grid=(M // _CR,),
        in_specs=[pl.BlockSpec((_CR, D), lambda i: (i, 0))],
        out_specs=pl.BlockSpec((_CR, D), lambda i: (i, 0)),
        out_shape=jax.ShapeDtypeStruct((M, D), jnp.float32),
    )(memory)


_BM = 2048


def _tc_body(mem_ref, msg_ref, w1m_ref, w1c_ref, w2_ref, out_ref):
    msg = msg_ref[...]
    mem = mem_ref[...]
    z = jnp.dot(msg, w1m_ref[...], preferred_element_type=jnp.float32)
    z = z + jnp.dot(mem, w1c_ref[...], preferred_element_type=jnp.float32)
    w = jnp.maximum(jnp.tanh(z), 0.0) * PARA
    u = jnp.tanh(jnp.dot(msg, w2_ref[...], preferred_element_type=jnp.float32))
    out_ref[...] = mem * (1.0 - w) + w * u


def _tc_update(mem_b, msgs, w1m, w1c, w2):
    return pl.pallas_call(
        _tc_body,
        grid=(B // _BM,),
        in_specs=[
            pl.BlockSpec((_BM, D), lambda i: (i, 0)),
            pl.BlockSpec((_BM, D), lambda i: (i, 0)),
            pl.BlockSpec((D, D), lambda i: (0, 0)),
            pl.BlockSpec((D, D), lambda i: (0, 0)),
            pl.BlockSpec((D, D), lambda i: (0, 0)),
        ],
        out_specs=pl.BlockSpec((_BM, D), lambda i: (i, 0)),
        out_shape=jax.ShapeDtypeStruct((B, D), jnp.float32),
    )(mem_b, msgs, w1m, w1c, w2)


def kernel(memory, unique_messages, W_lins, W_lin2, unique_node_ids, timestamps, last_update):
    sc_gather, sc_scatter = _sc_kernels()
    w1m = W_lins[:, :D].T
    w1c = W_lins[:, D:].T
    w2 = W_lin2.T

    lu_ref = jax.new_ref(last_update)
    mem_b = sc_gather(memory, unique_node_ids, timestamps, lu_ref)
    table = _tc_copy(memory)
    updated = _tc_update(mem_b, unique_messages, w1m, w1c, w2)

    tbl_ref = jax.new_ref(table)
    sc_scatter(updated, unique_node_ids, tbl_ref)
    return tbl_ref[...], lu_ref[...]
